# baseline (speedup 1.0000x reference)
import functools

import jax
import jax.numpy as jnp
from jax.experimental import pallas as pl
from jax.experimental.pallas import tpu as pltpu

_NUM_EXPERTS = 64
_TOP_K = 2
_B, _S, _D = 4, 8192, 768
_S_BLK = 512
_N_BLK = _S // _S_BLK


def _gate_kernel(x_ref, wh_ref, bh_ref, wo_ref, bo_ref,
                 w_out_ref, i_out_ref, acc_ref):
    step = pl.program_id(0)

    @pl.when(step == 0)
    def _init():
        acc_ref[...] = jnp.zeros_like(acc_ref)

    acc_ref[...] += jnp.sum(x_ref[...], axis=1)

    @pl.when(step == _N_BLK - 1)
    def _final():
        r = acc_ref[...] * (1.0 / _S)
        h = r @ wh_ref[...] + bh_ref[...]
        h = h * jax.nn.sigmoid(h)
        logits = h @ wo_ref[...] + bo_ref[...]

        iota = jax.lax.broadcasted_iota(jnp.int32, logits.shape, 1)
        m1 = jnp.max(logits, axis=-1, keepdims=True)
        i1 = jnp.min(jnp.where(logits == m1, iota, _NUM_EXPERTS),
                     axis=-1, keepdims=True)
        masked = jnp.where(iota == i1, -jnp.inf, logits)
        m2 = jnp.max(masked, axis=-1, keepdims=True)
        i2 = jnp.min(jnp.where(masked == m2, iota, _NUM_EXPERTS),
                     axis=-1, keepdims=True)

        e2 = jnp.exp(m2 - m1)
        denom = 1.0 + e2
        w_out_ref[...] = jnp.concatenate([1.0 / denom, e2 / denom], axis=-1)
        i_out_ref[...] = jnp.concatenate([i1, i2], axis=-1).astype(jnp.int32)


@jax.jit
def kernel(x, W_hidden, b_hidden, W_out, b_out):
    out_shapes = (
        jax.ShapeDtypeStruct((_B, _TOP_K), jnp.float32),
        jax.ShapeDtypeStruct((_B, _TOP_K), jnp.int32),
    )
    grid = (_N_BLK,)
    w, i = pl.pallas_call(
        _gate_kernel,
        grid=grid,
        in_specs=[
            pl.BlockSpec((_B, _S_BLK, _D), lambda s: (0, s, 0)),
            pl.BlockSpec((_D, _D), lambda s: (0, 0)),
            pl.BlockSpec((1, _D), lambda s: (0, 0)),
            pl.BlockSpec((_D, _NUM_EXPERTS), lambda s: (0, 0)),
            pl.BlockSpec((1, _NUM_EXPERTS), lambda s: (0, 0)),
        ],
        out_specs=(
            pl.BlockSpec((_B, _TOP_K), lambda s: (0, 0)),
            pl.BlockSpec((_B, _TOP_K), lambda s: (0, 0)),
        ),
        out_shape=out_shapes,
        scratch_shapes=[pltpu.VMEM((_B, _D), jnp.float32)],
        compiler_params=pltpu.CompilerParams(
            dimension_semantics=("arbitrary",),
        ),
    )(x, W_hidden, b_hidden.reshape(1, _D), W_out,
      b_out.reshape(1, _NUM_EXPERTS))
    return w, i
